# baseline (speedup 1.0000x reference)
import functools

import jax
import jax.numpy as jnp
from jax import lax
from jax.experimental import pallas as pl
from jax.experimental.pallas import tpu as pltpu

from jax.experimental.pallas import tpu_sc as plsc

_N = 10000
_E = 320000
_G = 64
_IN = 128
_HID = 64
_NC2 = 2

_NCORES = 2
_NSUB = 16
_NW = _NCORES * _NSUB
_B = 128
_STEPS = 80
_EPW = _STEPS * _B
_EPAD = _NW * _EPW
_R = 10112
_ZR = _R // _NSUB


@functools.lru_cache(maxsize=None)
def _build_sc_kernels():
    mesh = plsc.VectorSubcoreMesh(core_axis_name="c", subcore_axis_name="s",
                                  num_cores=_NCORES, num_subcores=_NSUB)
    params = pltpu.CompilerParams(use_tc_tiling_on_sc=False)

    @functools.partial(
        pl.kernel,
        out_type=(
            jax.ShapeDtypeStruct((_R, 16), jnp.float32),
            jax.ShapeDtypeStruct((_R, 16), jnp.float32),
        ),
        mesh=mesh,
        scratch_types=[
            pltpu.VMEM((_STEPS, _B), jnp.int32),
            pltpu.VMEM((_B, 16), jnp.float32),
            pltpu.VMEM_SHARED((_R, 16), jnp.float32),
            pltpu.SemaphoreType.DMA,
        ],
        compiler_params=params,
    )
    def deg_kernel(dst_hbm, zeros_hbm, ones_hbm, out0, out1,
                   idx_v, ones_v, acc_sh, sem):
        c = lax.axis_index("c")
        s = lax.axis_index("s")
        wid = s * _NCORES + c

        pltpu.sync_copy(ones_hbm, ones_v)
        pltpu.sync_copy(zeros_hbm, acc_sh.at[pl.ds(s * _ZR, _ZR)])
        plsc.subcore_barrier()

        pltpu.sync_copy(dst_hbm.at[wid], idx_v)

        def body(j, _):
            pltpu.async_copy(ones_v, acc_sh.at[idx_v.at[j]], sem, add=True)

            @pl.when(j >= 8)
            def _():
                pltpu.make_async_copy(ones_v, acc_sh.at[idx_v.at[0]],
                                      sem).wait()

            return 0

        lax.fori_loop(0, _STEPS, body, 0)

        def drain(j, _):
            pltpu.make_async_copy(ones_v, acc_sh.at[idx_v.at[0]], sem).wait()
            return 0

        lax.fori_loop(0, 8, drain, 0)
        plsc.subcore_barrier()

        @pl.when(c == 0)
        def _():
            pltpu.sync_copy(acc_sh.at[pl.ds(s * _ZR, _ZR)],
                            out0.at[pl.ds(s * _ZR, _ZR)])

        @pl.when(c == 1)
        def _():
            pltpu.sync_copy(acc_sh.at[pl.ds(s * _ZR, _ZR)],
                            out1.at[pl.ds(s * _ZR, _ZR)])

    @functools.partial(
        pl.kernel,
        out_type=(
            jax.ShapeDtypeStruct((_R, _HID), jnp.float32),
            jax.ShapeDtypeStruct((_R, _HID), jnp.float32),
        ),
        mesh=mesh,
        scratch_types=[
            pltpu.VMEM((_STEPS // 2, _B), jnp.int32),
            pltpu.VMEM((_STEPS // 2, _B), jnp.int32),
            pltpu.VMEM((_B, _HID), jnp.float32),
            pltpu.VMEM((_B, _HID), jnp.float32),
            pltpu.VMEM((_B, _HID), jnp.float32),
            pltpu.VMEM((_B, _HID), jnp.float32),
            pltpu.VMEM_SHARED((_R, _HID), jnp.float32),
            pltpu.VMEM_SHARED((_R, _HID), jnp.float32),
            pltpu.SemaphoreType.DMA,
            pltpu.SemaphoreType.DMA,
            pltpu.SemaphoreType.DMA,
            pltpu.SemaphoreType.DMA,
            pltpu.SemaphoreType.DMA,
            pltpu.SemaphoreType.DMA,
            pltpu.SemaphoreType.DMA,
            pltpu.SemaphoreType.DMA,
        ],
        compiler_params=params,
    )
    def edge_kernel(hs_hbm, src_hbm, dst_hbm, zeros_hbm, out0, out1,
                    src_v, dst_v, r0, r1, r2, r3, stage_sh, acc_sh,
                    g0, g1, g2, g3, s0, s1, s2, s3):
        c = lax.axis_index("c")
        s = lax.axis_index("s")
        wid = s * _NCORES + c
        sl = pl.ds(s * _ZR, _ZR)
        rows = (r0, r1, r2, r3)
        gsems = (g0, g1, g2, g3)
        ssems = (s0, s1, s2, s3)
        nbuf = 4

        pltpu.sync_copy(hs_hbm.at[sl], stage_sh.at[sl])

        @pl.when(c == 0)
        def _():
            pltpu.sync_copy(hs_hbm.at[sl], acc_sh.at[sl])

        @pl.when(c == 1)
        def _():
            pltpu.sync_copy(zeros_hbm, acc_sh.at[sl])

        plsc.subcore_barrier()

        half = _STEPS // 2
        for phase in range(2):
            pltpu.sync_copy(src_hbm.at[wid].at[pl.ds(phase * half, half)],
                            src_v)
            pltpu.sync_copy(dst_hbm.at[wid].at[pl.ds(phase * half, half)],
                            dst_v)
            for k in range(nbuf):
                pltpu.async_copy(stage_sh.at[src_v.at[k]], rows[k], gsems[k])

            def body(i, _):
                base = nbuf * i
                for k in range(nbuf):
                    pltpu.make_async_copy(stage_sh.at[src_v.at[0]], rows[k],
                                          gsems[k]).wait()
                    pltpu.async_copy(rows[k], acc_sh.at[dst_v.at[base + k]],
                                     ssems[k], add=True)
                for k in range(nbuf):
                    nj = base + nbuf + k

                    @pl.when(nj < half)
                    def _(k=k, nj=nj):
                        pltpu.make_async_copy(rows[k], acc_sh.at[dst_v.at[0]],
                                              ssems[k]).wait()
                        pltpu.async_copy(stage_sh.at[src_v.at[nj]], rows[k],
                                         gsems[k])

                return 0

            lax.fori_loop(0, half // nbuf, body, 0)
            for k in range(nbuf):
                pltpu.make_async_copy(rows[k], acc_sh.at[dst_v.at[0]],
                                      ssems[k]).wait()
        plsc.subcore_barrier()

        @pl.when(c == 0)
        def _():
            pltpu.sync_copy(acc_sh.at[pl.ds(s * _ZR, _ZR)],
                            out0.at[pl.ds(s * _ZR, _ZR)])

        @pl.when(c == 1)
        def _():
            pltpu.sync_copy(acc_sh.at[pl.ds(s * _ZR, _ZR)],
                            out1.at[pl.ds(s * _ZR, _ZR)])

    return deg_kernel, edge_kernel


def _mm_scale_body(x_ref, w_ref, d0_ref, d1_ref, hs_ref, dinv_ref):
    h = jnp.dot(x_ref[...], w_ref[...], preferred_element_type=jnp.float32)
    deg = d0_ref[0:_N, 0:1] + d1_ref[0:_N, 0:1] + 1.0
    dinv = lax.rsqrt(deg)
    dinv_ref[...] = dinv
    hs_ref[0:_N, :] = h * dinv
    hs_ref[_N:_R, :] = jnp.zeros((_R - _N, _HID), jnp.float32)


def _mid_body(a0_ref, a1_ref, dinv_ref, b_ref, w_ref, o_ref):
    z = (a0_ref[0:_N, :] + a1_ref[0:_N, :]) * dinv_ref[...] + b_ref[...]
    h = jnp.maximum(z, 0.0)
    o_ref[0:_N, :] = jnp.dot(h, w_ref[...],
                             preferred_element_type=jnp.float32) * dinv_ref[...]
    o_ref[_N:_R, :] = jnp.zeros((_R - _N, _HID), jnp.float32)


def _final_body(a0_ref, a1_ref, dinv_ref, b_ref, batch_ref,
                wc_ref, bc_ref, o_ref):
    z = (a0_ref[0:_N, :] + a1_ref[0:_N, :]) * dinv_ref[...] + b_ref[...]
    h = jnp.maximum(z, 0.0)
    gid = lax.broadcasted_iota(jnp.int32, (_N, _G), 1)
    mask = jnp.where(batch_ref[...] == gid, 1.0, 0.0)
    sums = lax.dot_general(mask, h, (((0,), (0,)), ((), ())),
                           preferred_element_type=jnp.float32)
    cnt = lax.dot_general(mask, jnp.ones((_N, 1), jnp.float32),
                          (((0,), (0,)), ((), ())),
                          preferred_element_type=jnp.float32)
    g = sums / jnp.maximum(cnt, 1.0)
    logits = jnp.dot(g, wc_ref[...],
                     preferred_element_type=jnp.float32) + bc_ref[...]
    m = jnp.max(logits, axis=1, keepdims=True)
    sh = logits - m
    lse = jnp.log(jnp.sum(jnp.exp(sh), axis=1, keepdims=True))
    o_ref[...] = sh - lse


def _tc_call(body, out_shape, *args):
    return pl.pallas_call(
        body,
        out_shape=out_shape,
    )(*args)


def kernel(x, edge_index, batch, W1, b1, W2, b2, Wc, bc):
    f32 = jnp.float32
    src = edge_index[0]
    dst = edge_index[1]
    pad = _EPAD - _E
    src3 = jnp.concatenate([src, jnp.zeros((pad,), jnp.int32)])
    src3 = src3.reshape(_NW, _STEPS, _B)
    dst3 = jnp.concatenate([dst, jnp.full((pad,), _N, jnp.int32)])
    dst3 = dst3.reshape(_NW, _STEPS, _B)
    batch2 = batch.reshape(_N, 1)
    b1r = b1.reshape(1, _HID)
    b2r = b2.reshape(1, _HID)
    bcr = bc.reshape(1, _NC2)

    deg_kernel, edge_kernel = _build_sc_kernels()
    zeros16 = jnp.zeros((_ZR, 16), f32)
    ones16 = jnp.ones((_B, 16), f32)
    zeros64 = jnp.zeros((_ZR, _HID), f32)

    deg0, deg1 = deg_kernel(dst3, zeros16, ones16)

    hs1, dinv = _tc_call(
        _mm_scale_body,
        (jax.ShapeDtypeStruct((_R, _HID), f32),
         jax.ShapeDtypeStruct((_N, 1), f32)),
        x, W1, deg0, deg1)

    a10, a11 = edge_kernel(hs1, src3, dst3, zeros64)

    hs2 = _tc_call(_mid_body, jax.ShapeDtypeStruct((_R, _HID), f32),
                   a10, a11, dinv, b1r, W2)

    a20, a21 = edge_kernel(hs2, src3, dst3, zeros64)

    out = _tc_call(_final_body, jax.ShapeDtypeStruct((_G, _NC2), f32),
                   a20, a21, dinv, b2r, batch2, Wc, bcr)
    return out


# speedup vs baseline: 1.1029x; 1.1029x over previous
import functools

import jax
import jax.numpy as jnp
from jax import lax
from jax.experimental import pallas as pl
from jax.experimental.pallas import tpu as pltpu

from jax.experimental.pallas import tpu_sc as plsc

_N = 10000
_E = 320000
_G = 64
_IN = 128
_HID = 64
_NC2 = 2

_NCORES = 2
_NSUB = 16
_NW = _NCORES * _NSUB
_B = 128
_STEPS = 80
_EPW = _STEPS * _B
_EPAD = _NW * _EPW
_R = 10112
_ZR = _R // _NSUB


@functools.lru_cache(maxsize=None)
def _build_sc_kernels():
    mesh = plsc.VectorSubcoreMesh(core_axis_name="c", subcore_axis_name="s",
                                  num_cores=_NCORES, num_subcores=_NSUB)
    params = pltpu.CompilerParams(use_tc_tiling_on_sc=False)

    @functools.partial(
        pl.kernel,
        out_type=(
            jax.ShapeDtypeStruct((_R, 16), jnp.float32),
            jax.ShapeDtypeStruct((_R, 16), jnp.float32),
        ),
        mesh=mesh,
        scratch_types=[
            pltpu.VMEM((_STEPS, _B), jnp.int32),
            pltpu.VMEM((_B, 16), jnp.float32),
            pltpu.VMEM_SHARED((_R, 16), jnp.float32),
            pltpu.SemaphoreType.DMA,
        ],
        compiler_params=params,
    )
    def deg_kernel(dst_hbm, zeros_hbm, ones_hbm, out0, out1,
                   idx_v, ones_v, acc_sh, sem):
        c = lax.axis_index("c")
        s = lax.axis_index("s")
        wid = s * _NCORES + c

        pltpu.sync_copy(ones_hbm, ones_v)
        pltpu.sync_copy(zeros_hbm, acc_sh.at[pl.ds(s * _ZR, _ZR)])
        plsc.subcore_barrier()

        pltpu.sync_copy(dst_hbm.at[wid], idx_v)

        def body(j, _):
            pltpu.async_copy(ones_v, acc_sh.at[idx_v.at[j]], sem, add=True)

            @pl.when(j >= 8)
            def _():
                pltpu.make_async_copy(ones_v, acc_sh.at[idx_v.at[0]],
                                      sem).wait()

            return 0

        lax.fori_loop(0, _STEPS, body, 0)

        def drain(j, _):
            pltpu.make_async_copy(ones_v, acc_sh.at[idx_v.at[0]], sem).wait()
            return 0

        lax.fori_loop(0, 8, drain, 0)
        plsc.subcore_barrier()

        @pl.when(c == 0)
        def _():
            pltpu.sync_copy(acc_sh.at[pl.ds(s * _ZR, _ZR)],
                            out0.at[pl.ds(s * _ZR, _ZR)])

        @pl.when(c == 1)
        def _():
            pltpu.sync_copy(acc_sh.at[pl.ds(s * _ZR, _ZR)],
                            out1.at[pl.ds(s * _ZR, _ZR)])

    @functools.partial(
        pl.kernel,
        out_type=(
            jax.ShapeDtypeStruct((_R, _HID), jnp.float32),
            jax.ShapeDtypeStruct((_R, _HID), jnp.float32),
        ),
        mesh=mesh,
        scratch_types=[
            pltpu.VMEM((_STEPS, _B), jnp.int32),
            pltpu.VMEM((_STEPS, _B), jnp.int32),
            pltpu.VMEM((_B, _HID), jnp.float32),
            pltpu.VMEM((_B, _HID), jnp.float32),
            pltpu.VMEM_SHARED((_R, _HID), jnp.float32),
            pltpu.VMEM_SHARED((_R, _HID), jnp.float32),
            pltpu.SemaphoreType.DMA,
            pltpu.SemaphoreType.DMA,
        ],
        compiler_params=params,
    )
    def edge_kernel(hs_hbm, src_hbm, dst_hbm, zeros_hbm, out0, out1,
                    src_v, dst_v, rows0_v, rows1_v, stage_sh, acc_sh,
                    sem0, sem1):
        c = lax.axis_index("c")
        s = lax.axis_index("s")
        wid = s * _NCORES + c
        sl = pl.ds(s * _ZR, _ZR)

        pltpu.sync_copy(hs_hbm.at[sl], stage_sh.at[sl])

        @pl.when(c == 0)
        def _():
            pltpu.sync_copy(hs_hbm.at[sl], acc_sh.at[sl])

        @pl.when(c == 1)
        def _():
            pltpu.sync_copy(zeros_hbm, acc_sh.at[sl])

        pltpu.sync_copy(src_hbm.at[wid], src_v)
        pltpu.sync_copy(dst_hbm.at[wid], dst_v)
        plsc.subcore_barrier()

        nhalf = _STEPS // 2
        pltpu.async_copy(stage_sh.at[src_v.at[0]], rows0_v, sem0)

        def body(i, _):
            j0 = 2 * i
            j1 = j0 + 1
            pltpu.async_copy(stage_sh.at[src_v.at[j1]], rows1_v, sem1)
            pltpu.make_async_copy(stage_sh.at[src_v.at[j0]], rows0_v,
                                  sem0).wait()
            pltpu.sync_copy(rows0_v, acc_sh.at[dst_v.at[j0]], add=True)

            @pl.when(i + 1 < nhalf)
            def _():
                pltpu.async_copy(stage_sh.at[src_v.at[j1 + 1]], rows0_v, sem0)

            pltpu.make_async_copy(stage_sh.at[src_v.at[j1]], rows1_v,
                                  sem1).wait()
            pltpu.sync_copy(rows1_v, acc_sh.at[dst_v.at[j1]], add=True)
            return 0

        lax.fori_loop(0, nhalf, body, 0)
        plsc.subcore_barrier()

        @pl.when(c == 0)
        def _():
            pltpu.sync_copy(acc_sh.at[pl.ds(s * _ZR, _ZR)],
                            out0.at[pl.ds(s * _ZR, _ZR)])

        @pl.when(c == 1)
        def _():
            pltpu.sync_copy(acc_sh.at[pl.ds(s * _ZR, _ZR)],
                            out1.at[pl.ds(s * _ZR, _ZR)])

    return deg_kernel, edge_kernel


def _mm_scale_body(x_ref, w_ref, d0_ref, d1_ref, hs_ref, dinv_ref):
    h = jnp.dot(x_ref[...], w_ref[...], preferred_element_type=jnp.float32)
    deg = d0_ref[0:_N, 0:1] + d1_ref[0:_N, 0:1] + 1.0
    dinv = lax.rsqrt(deg)
    dinv_ref[...] = dinv
    hs_ref[0:_N, :] = h * dinv
    hs_ref[_N:_R, :] = jnp.zeros((_R - _N, _HID), jnp.float32)


def _mid_body(a0_ref, a1_ref, dinv_ref, b_ref, w_ref, o_ref):
    z = (a0_ref[0:_N, :] + a1_ref[0:_N, :]) * dinv_ref[...] + b_ref[...]
    h = jnp.maximum(z, 0.0)
    o_ref[0:_N, :] = jnp.dot(h, w_ref[...],
                             preferred_element_type=jnp.float32) * dinv_ref[...]
    o_ref[_N:_R, :] = jnp.zeros((_R - _N, _HID), jnp.float32)


def _final_body(a0_ref, a1_ref, dinv_ref, b_ref, batch_ref,
                wc_ref, bc_ref, o_ref):
    z = (a0_ref[0:_N, :] + a1_ref[0:_N, :]) * dinv_ref[...] + b_ref[...]
    h = jnp.maximum(z, 0.0)
    gid = lax.broadcasted_iota(jnp.int32, (_N, _G), 1)
    mask = jnp.where(batch_ref[...] == gid, 1.0, 0.0)
    sums = lax.dot_general(mask, h, (((0,), (0,)), ((), ())),
                           preferred_element_type=jnp.float32)
    cnt = lax.dot_general(mask, jnp.ones((_N, 1), jnp.float32),
                          (((0,), (0,)), ((), ())),
                          preferred_element_type=jnp.float32)
    g = sums / jnp.maximum(cnt, 1.0)
    logits = jnp.dot(g, wc_ref[...],
                     preferred_element_type=jnp.float32) + bc_ref[...]
    m = jnp.max(logits, axis=1, keepdims=True)
    sh = logits - m
    lse = jnp.log(jnp.sum(jnp.exp(sh), axis=1, keepdims=True))
    o_ref[...] = sh - lse


def _tc_call(body, out_shape, *args):
    return pl.pallas_call(
        body,
        out_shape=out_shape,
    )(*args)


def kernel(x, edge_index, batch, W1, b1, W2, b2, Wc, bc):
    f32 = jnp.float32
    src = edge_index[0]
    dst = edge_index[1]
    pad = _EPAD - _E
    src3 = jnp.concatenate([src, jnp.zeros((pad,), jnp.int32)])
    src3 = src3.reshape(_NW, _STEPS, _B)
    dst3 = jnp.concatenate([dst, jnp.full((pad,), _N, jnp.int32)])
    dst3 = dst3.reshape(_NW, _STEPS, _B)
    batch2 = batch.reshape(_N, 1)
    b1r = b1.reshape(1, _HID)
    b2r = b2.reshape(1, _HID)
    bcr = bc.reshape(1, _NC2)

    deg_kernel, edge_kernel = _build_sc_kernels()
    zeros16 = jnp.zeros((_ZR, 16), f32)
    ones16 = jnp.ones((_B, 16), f32)
    zeros64 = jnp.zeros((_ZR, _HID), f32)

    deg0, deg1 = deg_kernel(dst3, zeros16, ones16)

    hs1, dinv = _tc_call(
        _mm_scale_body,
        (jax.ShapeDtypeStruct((_R, _HID), f32),
         jax.ShapeDtypeStruct((_N, 1), f32)),
        x, W1, deg0, deg1)

    a10, a11 = edge_kernel(hs1, src3, dst3, zeros64)

    hs2 = _tc_call(_mid_body, jax.ShapeDtypeStruct((_R, _HID), f32),
                   a10, a11, dinv, b1r, W2)

    a20, a21 = edge_kernel(hs2, src3, dst3, zeros64)

    out = _tc_call(_final_body, jax.ShapeDtypeStruct((_G, _NC2), f32),
                   a20, a21, dinv, b2r, batch2, Wc, bcr)
    return out


# speedup vs baseline: 1.1865x; 1.0758x over previous
import functools

import jax
import jax.numpy as jnp
from jax import lax
from jax.experimental import pallas as pl
from jax.experimental.pallas import tpu as pltpu

from jax.experimental.pallas import tpu_sc as plsc

_N = 10000
_E = 320000
_G = 64
_IN = 128
_HID = 64
_NC2 = 2

_NCORES = 2
_NSUB = 16
_NW = _NCORES * _NSUB
_B = 128
_STEPS = 81
_EPW = _STEPS * _B
_EPAD = _NW * _EPW
_R = 10112
_ZR = _R // _NSUB


@functools.lru_cache(maxsize=None)
def _build_sc_kernels():
    mesh = plsc.VectorSubcoreMesh(core_axis_name="c", subcore_axis_name="s",
                                  num_cores=_NCORES, num_subcores=_NSUB)
    params = pltpu.CompilerParams(use_tc_tiling_on_sc=False)

    @functools.partial(
        pl.kernel,
        out_type=(
            jax.ShapeDtypeStruct((_R, 16), jnp.float32),
            jax.ShapeDtypeStruct((_R, 16), jnp.float32),
        ),
        mesh=mesh,
        scratch_types=[
            pltpu.VMEM((_STEPS, _B), jnp.int32),
            pltpu.VMEM((_B, 16), jnp.float32),
            pltpu.VMEM_SHARED((_R, 16), jnp.float32),
            pltpu.SemaphoreType.DMA,
        ],
        compiler_params=params,
    )
    def deg_kernel(dst_hbm, zeros_hbm, ones_hbm, out0, out1,
                   idx_v, ones_v, acc_sh, sem):
        c = lax.axis_index("c")
        s = lax.axis_index("s")
        wid = s * _NCORES + c

        pltpu.sync_copy(ones_hbm, ones_v)
        pltpu.sync_copy(zeros_hbm, acc_sh.at[pl.ds(s * _ZR, _ZR)])
        plsc.subcore_barrier()

        pltpu.sync_copy(dst_hbm.at[wid], idx_v)

        def body(j, _):
            pltpu.async_copy(ones_v, acc_sh.at[idx_v.at[j]], sem, add=True)

            @pl.when(j >= 8)
            def _():
                pltpu.make_async_copy(ones_v, acc_sh.at[idx_v.at[0]],
                                      sem).wait()

            return 0

        lax.fori_loop(0, _STEPS, body, 0)

        def drain(j, _):
            pltpu.make_async_copy(ones_v, acc_sh.at[idx_v.at[0]], sem).wait()
            return 0

        lax.fori_loop(0, 8, drain, 0)
        plsc.subcore_barrier()

        @pl.when(c == 0)
        def _():
            pltpu.sync_copy(acc_sh.at[pl.ds(s * _ZR, _ZR)],
                            out0.at[pl.ds(s * _ZR, _ZR)])

        @pl.when(c == 1)
        def _():
            pltpu.sync_copy(acc_sh.at[pl.ds(s * _ZR, _ZR)],
                            out1.at[pl.ds(s * _ZR, _ZR)])

    @functools.partial(
        pl.kernel,
        out_type=(
            jax.ShapeDtypeStruct((_R, _HID), jnp.float32),
            jax.ShapeDtypeStruct((_R, _HID), jnp.float32),
        ),
        mesh=mesh,
        scratch_types=[
            pltpu.VMEM((_STEPS, _B), jnp.int32),
            pltpu.VMEM((_STEPS, _B), jnp.int32),
            pltpu.VMEM((_B, _HID), jnp.float32),
            pltpu.VMEM((_B, _HID), jnp.float32),
            pltpu.VMEM((_B, _HID), jnp.float32),
            pltpu.VMEM_SHARED((_R, _HID), jnp.float32),
            pltpu.VMEM_SHARED((_R, _HID), jnp.float32),
            pltpu.SemaphoreType.DMA,
            pltpu.SemaphoreType.DMA,
            pltpu.SemaphoreType.DMA,
            pltpu.SemaphoreType.DMA,
            pltpu.SemaphoreType.DMA,
            pltpu.SemaphoreType.DMA,
        ],
        compiler_params=params,
    )
    def edge_kernel(hs_hbm, src_hbm, dst_hbm, zeros_hbm, out0, out1,
                    src_v, dst_v, r0, r1, r2, stage_sh, acc_sh,
                    g0, g1, g2, s0, s1, s2):
        c = lax.axis_index("c")
        s = lax.axis_index("s")
        wid = s * _NCORES + c
        sl = pl.ds(s * _ZR, _ZR)
        rows = (r0, r1, r2)
        gsems = (g0, g1, g2)
        ssems = (s0, s1, s2)

        pltpu.sync_copy(hs_hbm.at[sl], stage_sh.at[sl])

        @pl.when(c == 0)
        def _():
            pltpu.sync_copy(hs_hbm.at[sl], acc_sh.at[sl])

        @pl.when(c == 1)
        def _():
            pltpu.sync_copy(zeros_hbm, acc_sh.at[sl])

        pltpu.sync_copy(src_hbm.at[wid], src_v)
        pltpu.sync_copy(dst_hbm.at[wid], dst_v)
        plsc.subcore_barrier()

        pltpu.async_copy(stage_sh.at[src_v.at[0]], rows[0], gsems[0])
        pltpu.async_copy(stage_sh.at[src_v.at[1]], rows[1], gsems[1])

        def body(i, _):
            for k in range(3):
                j = 3 * i + k
                kp = (k + 2) % 3
                pltpu.make_async_copy(stage_sh.at[src_v.at[0]], rows[k],
                                      gsems[k]).wait()
                pltpu.async_copy(rows[k], acc_sh.at[dst_v.at[j]],
                                 ssems[k], add=True)

                @pl.when(j + 2 < _STEPS)
                def _(j=j, kp=kp):
                    @pl.when(j >= 1)
                    def _():
                        pltpu.make_async_copy(rows[kp],
                                              acc_sh.at[dst_v.at[0]],
                                              ssems[kp]).wait()

                    pltpu.async_copy(stage_sh.at[src_v.at[j + 2]], rows[kp],
                                     gsems[kp])

            return 0

        lax.fori_loop(0, _STEPS // 3, body, 0)
        for k in range(3):
            pltpu.make_async_copy(rows[k], acc_sh.at[dst_v.at[0]],
                                  ssems[k]).wait()
        plsc.subcore_barrier()

        @pl.when(c == 0)
        def _():
            pltpu.sync_copy(acc_sh.at[pl.ds(s * _ZR, _ZR)],
                            out0.at[pl.ds(s * _ZR, _ZR)])

        @pl.when(c == 1)
        def _():
            pltpu.sync_copy(acc_sh.at[pl.ds(s * _ZR, _ZR)],
                            out1.at[pl.ds(s * _ZR, _ZR)])

    return deg_kernel, edge_kernel


def _mm_scale_body(x_ref, w_ref, d0_ref, d1_ref, hs_ref, dinv_ref):
    h = jnp.dot(x_ref[...], w_ref[...], preferred_element_type=jnp.float32)
    deg = d0_ref[0:_N, 0:1] + d1_ref[0:_N, 0:1] + 1.0
    dinv = lax.rsqrt(deg)
    dinv_ref[...] = dinv
    hs_ref[0:_N, :] = h * dinv
    hs_ref[_N:_R, :] = jnp.zeros((_R - _N, _HID), jnp.float32)


def _mid_body(a0_ref, a1_ref, dinv_ref, b_ref, w_ref, o_ref):
    z = (a0_ref[0:_N, :] + a1_ref[0:_N, :]) * dinv_ref[...] + b_ref[...]
    h = jnp.maximum(z, 0.0)
    o_ref[0:_N, :] = jnp.dot(h, w_ref[...],
                             preferred_element_type=jnp.float32) * dinv_ref[...]
    o_ref[_N:_R, :] = jnp.zeros((_R - _N, _HID), jnp.float32)


def _final_body(a0_ref, a1_ref, dinv_ref, b_ref, batch_ref,
                wc_ref, bc_ref, o_ref):
    z = (a0_ref[0:_N, :] + a1_ref[0:_N, :]) * dinv_ref[...] + b_ref[...]
    h = jnp.maximum(z, 0.0)
    gid = lax.broadcasted_iota(jnp.int32, (_N, _G), 1)
    mask = jnp.where(batch_ref[...] == gid, 1.0, 0.0)
    sums = lax.dot_general(mask, h, (((0,), (0,)), ((), ())),
                           preferred_element_type=jnp.float32)
    cnt = lax.dot_general(mask, jnp.ones((_N, 1), jnp.float32),
                          (((0,), (0,)), ((), ())),
                          preferred_element_type=jnp.float32)
    g = sums / jnp.maximum(cnt, 1.0)
    logits = jnp.dot(g, wc_ref[...],
                     preferred_element_type=jnp.float32) + bc_ref[...]
    m = jnp.max(logits, axis=1, keepdims=True)
    sh = logits - m
    lse = jnp.log(jnp.sum(jnp.exp(sh), axis=1, keepdims=True))
    o_ref[...] = sh - lse


def _tc_call(body, out_shape, *args):
    return pl.pallas_call(
        body,
        out_shape=out_shape,
    )(*args)


def kernel(x, edge_index, batch, W1, b1, W2, b2, Wc, bc):
    f32 = jnp.float32
    src = edge_index[0]
    dst = edge_index[1]
    pad = _EPAD - _E
    src3 = jnp.concatenate([src, jnp.zeros((pad,), jnp.int32)])
    src3 = src3.reshape(_NW, _STEPS, _B)
    dst3 = jnp.concatenate([dst, jnp.full((pad,), _N, jnp.int32)])
    dst3 = dst3.reshape(_NW, _STEPS, _B)
    batch2 = batch.reshape(_N, 1)
    b1r = b1.reshape(1, _HID)
    b2r = b2.reshape(1, _HID)
    bcr = bc.reshape(1, _NC2)

    deg_kernel, edge_kernel = _build_sc_kernels()
    zeros16 = jnp.zeros((_ZR, 16), f32)
    ones16 = jnp.ones((_B, 16), f32)
    zeros64 = jnp.zeros((_ZR, _HID), f32)

    deg0, deg1 = deg_kernel(dst3, zeros16, ones16)

    hs1, dinv = _tc_call(
        _mm_scale_body,
        (jax.ShapeDtypeStruct((_R, _HID), f32),
         jax.ShapeDtypeStruct((_N, 1), f32)),
        x, W1, deg0, deg1)

    a10, a11 = edge_kernel(hs1, src3, dst3, zeros64)

    hs2 = _tc_call(_mid_body, jax.ShapeDtypeStruct((_R, _HID), f32),
                   a10, a11, dinv, b1r, W2)

    a20, a21 = edge_kernel(hs2, src3, dst3, zeros64)

    out = _tc_call(_final_body, jax.ShapeDtypeStruct((_G, _NC2), f32),
                   a20, a21, dinv, b2r, batch2, Wc, bcr)
    return out


# speedup vs baseline: 1.3389x; 1.1284x over previous
import functools

import jax
import jax.numpy as jnp
from jax import lax
from jax.experimental import pallas as pl
from jax.experimental.pallas import tpu as pltpu

from jax.experimental.pallas import tpu_sc as plsc

_N = 10000
_E = 320000
_G = 64
_IN = 128
_HID = 64
_NC2 = 2

_NCORES = 2
_NSUB = 16
_NW = _NCORES * _NSUB
_B = 128
_NBLK = _E // _B
_MS = _NBLK // _NW
_XTRA = _NBLK - _MS * _NW
_R = 10112
_ZR = _R // _NSUB


@functools.lru_cache(maxsize=None)
def _build_sc_kernels():
    mesh = plsc.VectorSubcoreMesh(core_axis_name="c", subcore_axis_name="s",
                                  num_cores=_NCORES, num_subcores=_NSUB)
    params = pltpu.CompilerParams(use_tc_tiling_on_sc=False)

    @functools.partial(
        pl.kernel,
        out_type=(
            jax.ShapeDtypeStruct((_R, 16), jnp.float32),
            jax.ShapeDtypeStruct((_R, 16), jnp.float32),
        ),
        mesh=mesh,
        scratch_types=[
            pltpu.VMEM((_MS + 1, _B), jnp.int32),
            pltpu.VMEM((_B, 16), jnp.float32),
            pltpu.VMEM_SHARED((_R, 16), jnp.float32),
            pltpu.SemaphoreType.DMA,
        ],
        compiler_params=params,
    )
    def deg_kernel(ei_hbm, zeros_hbm, ones_hbm, out0, out1,
                   idx_v, ones_v, acc_sh, sem):
        c = lax.axis_index("c")
        s = lax.axis_index("s")
        wid = s * _NCORES + c

        pltpu.sync_copy(ones_hbm, ones_v)
        pltpu.sync_copy(zeros_hbm, acc_sh.at[pl.ds(s * _ZR, _ZR)])
        plsc.subcore_barrier()

        dst_hbm = ei_hbm.at[1]
        pltpu.sync_copy(dst_hbm.at[pl.ds(wid * _MS, _MS)],
                        idx_v.at[pl.ds(0, _MS)])

        @pl.when(wid < _XTRA)
        def _():
            pltpu.sync_copy(dst_hbm.at[pl.ds(_NW * _MS + wid, 1)],
                            idx_v.at[pl.ds(_MS, 1)])

        def body(j, _):
            pltpu.async_copy(ones_v, acc_sh.at[idx_v.at[j]], sem, add=True)

            @pl.when(j >= 8)
            def _():
                pltpu.make_async_copy(ones_v, acc_sh.at[idx_v.at[0]],
                                      sem).wait()

            return 0

        lax.fori_loop(0, _MS, body, 0)

        @pl.when(wid < _XTRA)
        def _():
            pltpu.async_copy(ones_v, acc_sh.at[idx_v.at[_MS]], sem, add=True)
            pltpu.make_async_copy(ones_v, acc_sh.at[idx_v.at[0]], sem).wait()

        def drain(j, _):
            pltpu.make_async_copy(ones_v, acc_sh.at[idx_v.at[0]], sem).wait()
            return 0

        lax.fori_loop(0, 8, drain, 0)
        plsc.subcore_barrier()

        @pl.when(c == 0)
        def _():
            pltpu.sync_copy(acc_sh.at[pl.ds(s * _ZR, _ZR)],
                            out0.at[pl.ds(s * _ZR, _ZR)])

        @pl.when(c == 1)
        def _():
            pltpu.sync_copy(acc_sh.at[pl.ds(s * _ZR, _ZR)],
                            out1.at[pl.ds(s * _ZR, _ZR)])

    @functools.partial(
        pl.kernel,
        out_type=(
            jax.ShapeDtypeStruct((_R, _HID), jnp.float32),
            jax.ShapeDtypeStruct((_R, _HID), jnp.float32),
        ),
        mesh=mesh,
        scratch_types=[
            pltpu.VMEM((_MS + 1, _B), jnp.int32),
            pltpu.VMEM((_MS + 1, _B), jnp.int32),
            pltpu.VMEM((_B, _HID), jnp.float32),
            pltpu.VMEM((_B, _HID), jnp.float32),
            pltpu.VMEM((_B, _HID), jnp.float32),
            pltpu.VMEM_SHARED((_R, _HID), jnp.float32),
            pltpu.VMEM_SHARED((_R, _HID), jnp.float32),
            pltpu.SemaphoreType.DMA,
            pltpu.SemaphoreType.DMA,
            pltpu.SemaphoreType.DMA,
            pltpu.SemaphoreType.DMA,
            pltpu.SemaphoreType.DMA,
            pltpu.SemaphoreType.DMA,
        ],
        compiler_params=params,
    )
    def edge_kernel(hs_hbm, ei_hbm, zeros_hbm, out0, out1,
                    src_v, dst_v, r0, r1, r2, stage_sh, acc_sh,
                    g0, g1, g2, s0, s1, s2):
        c = lax.axis_index("c")
        s = lax.axis_index("s")
        wid = s * _NCORES + c
        sl = pl.ds(s * _ZR, _ZR)
        rows = (r0, r1, r2)
        gsems = (g0, g1, g2)
        ssems = (s0, s1, s2)

        pltpu.sync_copy(hs_hbm.at[sl], stage_sh.at[sl])

        @pl.when(c == 0)
        def _():
            pltpu.sync_copy(hs_hbm.at[sl], acc_sh.at[sl])

        @pl.when(c == 1)
        def _():
            pltpu.sync_copy(zeros_hbm, acc_sh.at[sl])

        src_hbm = ei_hbm.at[0]
        dst_hbm = ei_hbm.at[1]
        pltpu.sync_copy(src_hbm.at[pl.ds(wid * _MS, _MS)],
                        src_v.at[pl.ds(0, _MS)])
        pltpu.sync_copy(dst_hbm.at[pl.ds(wid * _MS, _MS)],
                        dst_v.at[pl.ds(0, _MS)])

        @pl.when(wid < _XTRA)
        def _():
            pltpu.sync_copy(src_hbm.at[pl.ds(_NW * _MS + wid, 1)],
                            src_v.at[pl.ds(_MS, 1)])
            pltpu.sync_copy(dst_hbm.at[pl.ds(_NW * _MS + wid, 1)],
                            dst_v.at[pl.ds(_MS, 1)])

        plsc.subcore_barrier()

        pltpu.async_copy(stage_sh.at[src_v.at[0]], rows[0], gsems[0])
        pltpu.async_copy(stage_sh.at[src_v.at[1]], rows[1], gsems[1])

        def body(i, _):
            for k in range(3):
                j = 3 * i + k
                kp = (k + 2) % 3
                pltpu.make_async_copy(stage_sh.at[src_v.at[0]], rows[k],
                                      gsems[k]).wait()
                pltpu.async_copy(rows[k], acc_sh.at[dst_v.at[j]],
                                 ssems[k], add=True)

                @pl.when(j + 2 < _MS)
                def _(j=j, kp=kp):
                    @pl.when(j >= 1)
                    def _():
                        pltpu.make_async_copy(rows[kp],
                                              acc_sh.at[dst_v.at[0]],
                                              ssems[kp]).wait()

                    pltpu.async_copy(stage_sh.at[src_v.at[j + 2]], rows[kp],
                                     gsems[kp])

            return 0

        lax.fori_loop(0, _MS // 3, body, 0)
        for k in range(3):
            pltpu.make_async_copy(rows[k], acc_sh.at[dst_v.at[0]],
                                  ssems[k]).wait()

        @pl.when(wid < _XTRA)
        def _():
            pltpu.async_copy(stage_sh.at[src_v.at[_MS]], rows[0], gsems[0])
            pltpu.make_async_copy(stage_sh.at[src_v.at[0]], rows[0],
                                  gsems[0]).wait()
            pltpu.sync_copy(rows[0], acc_sh.at[dst_v.at[_MS]], add=True)

        plsc.subcore_barrier()

        @pl.when(c == 0)
        def _():
            pltpu.sync_copy(acc_sh.at[pl.ds(s * _ZR, _ZR)],
                            out0.at[pl.ds(s * _ZR, _ZR)])

        @pl.when(c == 1)
        def _():
            pltpu.sync_copy(acc_sh.at[pl.ds(s * _ZR, _ZR)],
                            out1.at[pl.ds(s * _ZR, _ZR)])

    return deg_kernel, edge_kernel


def _mm_scale_body(x_ref, w_ref, d0_ref, d1_ref, hs_ref, dinv_ref):
    h = jnp.dot(x_ref[...], w_ref[...], preferred_element_type=jnp.float32)
    deg = d0_ref[0:_N, 0:1] + d1_ref[0:_N, 0:1] + 1.0
    dinv = lax.rsqrt(deg)
    dinv_ref[...] = dinv
    hs_ref[0:_N, :] = h * dinv
    hs_ref[_N:_R, :] = jnp.zeros((_R - _N, _HID), jnp.float32)


def _mid_body(a0_ref, a1_ref, dinv_ref, b_ref, w_ref, o_ref):
    z = (a0_ref[0:_N, :] + a1_ref[0:_N, :]) * dinv_ref[...] + b_ref[...]
    h = jnp.maximum(z, 0.0)
    o_ref[0:_N, :] = jnp.dot(h, w_ref[...],
                             preferred_element_type=jnp.float32) * dinv_ref[...]
    o_ref[_N:_R, :] = jnp.zeros((_R - _N, _HID), jnp.float32)


def _final_body(a0_ref, a1_ref, dinv_ref, b_ref, batch_ref,
                wc_ref, bc_ref, o_ref):
    z = (a0_ref[0:_N, :] + a1_ref[0:_N, :]) * dinv_ref[...] + b_ref[...]
    h = jnp.maximum(z, 0.0)
    gid = lax.broadcasted_iota(jnp.int32, (_N, _G), 1)
    mask = jnp.where(batch_ref[...] == gid, 1.0, 0.0)
    sums = lax.dot_general(mask, h, (((0,), (0,)), ((), ())),
                           preferred_element_type=jnp.float32)
    cnt = lax.dot_general(mask, jnp.ones((_N, 1), jnp.float32),
                          (((0,), (0,)), ((), ())),
                          preferred_element_type=jnp.float32)
    g = sums / jnp.maximum(cnt, 1.0)
    logits = jnp.dot(g, wc_ref[...],
                     preferred_element_type=jnp.float32) + bc_ref[...]
    m = jnp.max(logits, axis=1, keepdims=True)
    sh = logits - m
    lse = jnp.log(jnp.sum(jnp.exp(sh), axis=1, keepdims=True))
    o_ref[...] = sh - lse


def _tc_call(body, out_shape, *args):
    return pl.pallas_call(
        body,
        out_shape=out_shape,
    )(*args)


def kernel(x, edge_index, batch, W1, b1, W2, b2, Wc, bc):
    f32 = jnp.float32
    ei3 = edge_index.reshape(2, _NBLK, _B)
    batch2 = batch.reshape(_N, 1)
    b1r = b1.reshape(1, _HID)
    b2r = b2.reshape(1, _HID)
    bcr = bc.reshape(1, _NC2)

    deg_kernel, edge_kernel = _build_sc_kernels()
    zeros16 = jnp.zeros((_ZR, 16), f32)
    ones16 = jnp.ones((_B, 16), f32)
    zeros64 = jnp.zeros((_ZR, _HID), f32)

    deg0, deg1 = deg_kernel(ei3, zeros16, ones16)

    hs1, dinv = _tc_call(
        _mm_scale_body,
        (jax.ShapeDtypeStruct((_R, _HID), f32),
         jax.ShapeDtypeStruct((_N, 1), f32)),
        x, W1, deg0, deg1)

    a10, a11 = edge_kernel(hs1, ei3, zeros64)

    hs2 = _tc_call(_mid_body, jax.ShapeDtypeStruct((_R, _HID), f32),
                   a10, a11, dinv, b1r, W2)

    a20, a21 = edge_kernel(hs2, ei3, zeros64)

    out = _tc_call(_final_body, jax.ShapeDtypeStruct((_G, _NC2), f32),
                   a20, a21, dinv, b2r, batch2, Wc, bcr)
    return out
